# Initial kernel scaffold; baseline (speedup 1.0000x reference)
import jax
import jax.numpy as jnp
from jax.experimental import pallas as pl


def kernel(a, b, c, y):
    raise NotImplementedError("write your pallas kernel here")


# speedup vs baseline: 5.7456x; 5.7456x over previous
import functools

import jax
import jax.numpy as jnp
from jax import lax
from jax.experimental import pallas as pl
from jax.experimental.pallas import tpu as pltpu
from jax.experimental.pallas import tpu_sc as plsc

_NUM_CORES = 2
_NUM_SUBCORES = 16
_NUM_WORKERS = _NUM_CORES * _NUM_SUBCORES


def _make_body(n, d):
    row_words = (n - 1) * d

    def _sampler_body(yy_hbm, out_hbm, yy_v, sem):
        rows_per_worker = n // _NUM_WORKERS
        cid = lax.axis_index("c")
        sid = lax.axis_index("s")
        wid = sid * _NUM_CORES + cid
        base = wid * rows_per_worker

        pltpu.sync_copy(yy_hbm, yy_v)

        def _copy(j):
            i = base + j
            return pltpu.make_async_copy(
                yy_v.at[pl.ds((i + 1) * d, row_words)],
                out_hbm.at[pl.ds(i * row_words, row_words)],
                sem,
            )

        def _issue(j, carry):
            _copy(j).start()
            return carry

        def _drain(j, carry):
            _copy(j).wait()
            return carry

        lax.fori_loop(0, rows_per_worker, _issue, 0)
        lax.fori_loop(0, rows_per_worker, _drain, 0)

    return _sampler_body


def kernel(a, b, c, y):
    n, d = y.shape
    yy = jnp.concatenate([y, y], axis=0).reshape(2 * n * d)
    mesh = plsc.VectorSubcoreMesh(core_axis_name="c", subcore_axis_name="s")
    run = pl.kernel(
        _make_body(n, d),
        mesh=mesh,
        out_type=jax.ShapeDtypeStruct((n * (n - 1) * d,), jnp.float32),
        scratch_types=[
            pltpu.VMEM((2 * n * d,), jnp.float32),
            pltpu.SemaphoreType.DMA,
        ],
    )
    return run(yy).reshape(n, n - 1, d)
